# baseline (speedup 1.0000x reference)
import functools

import jax
import jax.numpy as jnp
from jax import lax
from jax.experimental import pallas as pl
from jax.experimental.pallas import tpu as pltpu
from jax.experimental.pallas import tpu_sc as plsc

H = 128
NUM_LAYER = 2
NUM_GRAPHS = 64

N_OBJ = 50000
N_P1 = 200000
N_P2 = 200000

NOP = 50176
P1P = 200704
P2P = 200704
E1P = P1P
E2P = 2 * P2P

NW = 32
NSC = 2
NTILE = 16
BLK = 128
CW = 16
STRIPE = NOP // NTILE


def _mlp_body(x_ref, w1_ref, b1_ref, w2_ref, b2_ref, o_ref):
    x = x_ref[...]
    h = jnp.maximum(
        jnp.dot(x, w1_ref[...], preferred_element_type=jnp.float32)
        + b1_ref[...], 0.0)
    o_ref[...] = (
        jnp.dot(h, w2_ref[...], preferred_element_type=jnp.float32)
        + b2_ref[...])


def _mlp_obj_body(x_ref, w1_ref, b1_ref, w2_ref, b2_ref, o_ref, obf_ref):
    x = x_ref[...]
    h = jnp.maximum(
        jnp.dot(x, w1_ref[...], preferred_element_type=jnp.float32)
        + b1_ref[...], 0.0)
    o = (jnp.dot(h, w2_ref[...], preferred_element_type=jnp.float32)
         + b2_ref[...])
    o_ref[...] = o
    obf_ref[...] = o.astype(jnp.bfloat16)


def _mlp_add_body(x1_ref, x2_ref, w1_ref, b1_ref, w2_ref, b2_ref, o_ref):
    x = x1_ref[...] + x2_ref[...].astype(jnp.float32)
    h = jnp.maximum(
        jnp.dot(x, w1_ref[...], preferred_element_type=jnp.float32)
        + b1_ref[...], 0.0)
    o_ref[...] = (
        jnp.dot(h, w2_ref[...], preferred_element_type=jnp.float32)
        + b2_ref[...])


def _mlp_split_body(x_ref, w1_ref, b1_ref, w2_ref, b2_ref, oa_ref, ob_ref):
    x = x_ref[...]
    h = jnp.maximum(
        jnp.dot(x, w1_ref[...], preferred_element_type=jnp.float32)
        + b1_ref[...], 0.0)
    o = (jnp.dot(h, w2_ref[...], preferred_element_type=jnp.float32)
         + b2_ref[...])
    oa_ref[...] = o[:, :H]
    ob_ref[...] = o[:, H:]


def _mlp_add2_body(x1a_ref, x1b_ref, x2a_ref, x2b_ref, w1_ref, b1_ref,
                   w2_ref, b2_ref, oa_ref, ob_ref):
    x = jnp.concatenate(
        [x1a_ref[...] + x2a_ref[...].astype(jnp.float32),
         x1b_ref[...] + x2b_ref[...].astype(jnp.float32)], axis=1)
    h = jnp.maximum(
        jnp.dot(x, w1_ref[...], preferred_element_type=jnp.float32)
        + b1_ref[...], 0.0)
    o = (jnp.dot(h, w2_ref[...], preferred_element_type=jnp.float32)
         + b2_ref[...])
    oa_ref[...] = o[:, :H]
    ob_ref[...] = o[:, H:]


def _mlp_cat_body(x1_ref, x2_ref, w1a_ref, w1b_ref, b1_ref, w2_ref, b2_ref,
                  o_ref, obf_ref):
    z = (jnp.dot(x1_ref[...], w1a_ref[...], preferred_element_type=jnp.float32)
         + jnp.dot(x2_ref[...], w1b_ref[...],
                   preferred_element_type=jnp.float32)
         + b1_ref[...])
    h = jnp.maximum(z, 0.0)
    o = (jnp.dot(h, w2_ref[...], preferred_element_type=jnp.float32)
         + b2_ref[...])
    o_ref[...] = o
    obf_ref[...] = o.astype(jnp.bfloat16)


def _const_spec(shape):
    return pl.BlockSpec(shape, lambda i: (0,) * len(shape))


def _mlp(x, w1, b1, w2, b2, bn):
    n, k = x.shape
    h1 = w1.shape[1]
    h2 = w2.shape[1]
    return pl.pallas_call(
        _mlp_body,
        grid=(n // bn,),
        in_specs=[
            pl.BlockSpec((bn, k), lambda i: (i, 0)),
            _const_spec((k, h1)),
            _const_spec((1, h1)),
            _const_spec((h1, h2)),
            _const_spec((1, h2)),
        ],
        out_specs=pl.BlockSpec((bn, h2), lambda i: (i, 0)),
        out_shape=jax.ShapeDtypeStruct((n, h2), jnp.float32),
    )(x, w1, b1.reshape(1, -1), w2, b2.reshape(1, -1))


def _mlp_obj(x, w1, b1, w2, b2, bn):
    n, k = x.shape
    h1 = w1.shape[1]
    h2 = w2.shape[1]
    return pl.pallas_call(
        _mlp_obj_body,
        grid=(n // bn,),
        in_specs=[
            pl.BlockSpec((bn, k), lambda i: (i, 0)),
            _const_spec((k, h1)),
            _const_spec((1, h1)),
            _const_spec((h1, h2)),
            _const_spec((1, h2)),
        ],
        out_specs=[pl.BlockSpec((bn, h2), lambda i: (i, 0)),
                   pl.BlockSpec((bn, h2), lambda i: (i, 0))],
        out_shape=[jax.ShapeDtypeStruct((n, h2), jnp.float32),
                   jax.ShapeDtypeStruct((n, h2), jnp.bfloat16)],
    )(x, w1, b1.reshape(1, -1), w2, b2.reshape(1, -1))


def _mlp_add(x1, x2, w1, b1, w2, b2, bn):
    n, k = x1.shape
    h1 = w1.shape[1]
    h2 = w2.shape[1]
    return pl.pallas_call(
        _mlp_add_body,
        grid=(n // bn,),
        in_specs=[
            pl.BlockSpec((bn, k), lambda i: (i, 0)),
            pl.BlockSpec((bn, k), lambda i: (i, 0)),
            _const_spec((k, h1)),
            _const_spec((1, h1)),
            _const_spec((h1, h2)),
            _const_spec((1, h2)),
        ],
        out_specs=pl.BlockSpec((bn, h2), lambda i: (i, 0)),
        out_shape=jax.ShapeDtypeStruct((n, h2), jnp.float32),
    )(x1, x2, w1, b1.reshape(1, -1), w2, b2.reshape(1, -1))


def _mlp_split(x, w1, b1, w2, b2, bn):
    n, k = x.shape
    h1 = w1.shape[1]
    return pl.pallas_call(
        _mlp_split_body,
        grid=(n // bn,),
        in_specs=[
            pl.BlockSpec((bn, k), lambda i: (i, 0)),
            _const_spec((k, h1)),
            _const_spec((1, h1)),
            _const_spec((h1, 2 * H)),
            _const_spec((1, 2 * H)),
        ],
        out_specs=[pl.BlockSpec((bn, H), lambda i: (i, 0)),
                   pl.BlockSpec((bn, H), lambda i: (i, 0))],
        out_shape=[jax.ShapeDtypeStruct((n, H), jnp.float32),
                   jax.ShapeDtypeStruct((n, H), jnp.float32)],
    )(x, w1, b1.reshape(1, -1), w2, b2.reshape(1, -1))


def _mlp_add2(x1a, x1b, x2a, x2b, w1, b1, w2, b2, bn):
    n = x1a.shape[0]
    h1 = w1.shape[1]
    return pl.pallas_call(
        _mlp_add2_body,
        grid=(n // bn,),
        in_specs=[
            pl.BlockSpec((bn, H), lambda i: (i, 0)),
            pl.BlockSpec((bn, H), lambda i: (i, 0)),
            pl.BlockSpec((bn, H), lambda i: (i, 0)),
            pl.BlockSpec((bn, H), lambda i: (i, 0)),
            _const_spec((2 * H, h1)),
            _const_spec((1, h1)),
            _const_spec((h1, 2 * H)),
            _const_spec((1, 2 * H)),
        ],
        out_specs=[pl.BlockSpec((bn, H), lambda i: (i, 0)),
                   pl.BlockSpec((bn, H), lambda i: (i, 0))],
        out_shape=[jax.ShapeDtypeStruct((n, H), jnp.float32),
                   jax.ShapeDtypeStruct((n, H), jnp.float32)],
    )(x1a, x1b, x2a, x2b, w1, b1.reshape(1, -1), w2, b2.reshape(1, -1))


def _mlp_cat(x1, x2, w1a, w1b, b1, w2, b2, bn):
    n, k = x1.shape
    h1 = w1a.shape[1]
    h2 = w2.shape[1]
    return pl.pallas_call(
        _mlp_cat_body,
        grid=(n // bn,),
        in_specs=[
            pl.BlockSpec((bn, k), lambda i: (i, 0)),
            pl.BlockSpec((bn, k), lambda i: (i, 0)),
            _const_spec((k, h1)),
            _const_spec((k, h1)),
            _const_spec((1, h1)),
            _const_spec((h1, h2)),
            _const_spec((1, h2)),
        ],
        out_specs=[pl.BlockSpec((bn, h2), lambda i: (i, 0)),
                   pl.BlockSpec((bn, h2), lambda i: (i, 0))],
        out_shape=[jax.ShapeDtypeStruct((n, h2), jnp.float32),
                   jax.ShapeDtypeStruct((n, h2), jnp.bfloat16)],
    )(x1, x2, w1a, w1b, b1.reshape(1, -1), w2, b2.reshape(1, -1))


def _pool_body(x_ref, bid_ref, o_ref):
    ids = bid_ref[0, 0, :]
    bn = ids.shape[0]
    onehot = (ids.reshape(bn, 1)
              == lax.broadcasted_iota(jnp.int32, (1, NUM_GRAPHS), 1)
              ).astype(jnp.float32)
    contrib = lax.dot_general(
        onehot, x_ref[...], (((0,), (0,)), ((), ())),
        preferred_element_type=jnp.float32)

    @pl.when(pl.program_id(0) == 0)
    def _():
        o_ref[...] = jnp.zeros_like(o_ref)

    o_ref[...] += contrib


def _pool(x, bid3d, bn):
    n = x.shape[0]
    return pl.pallas_call(
        _pool_body,
        grid=(n // bn,),
        in_specs=[
            pl.BlockSpec((bn, H), lambda i: (i, 0)),
            pl.BlockSpec((1, 1, bn), lambda i: (i, 0, 0)),
        ],
        out_specs=pl.BlockSpec((NUM_GRAPHS, H), lambda i: (0, 0)),
        out_shape=jax.ShapeDtypeStruct((NUM_GRAPHS, H), jnp.float32),
    )(x, bid3d)


GG = 3
GNB = E1P // (NW * BLK)
GNG = GNB // GG
GTAIL = GNB - GNG * GG


@functools.lru_cache(maxsize=None)
def _make_gather():
    mesh = plsc.VectorSubcoreMesh(core_axis_name="c", subcore_axis_name="s")
    shp = jax.ShapeDtypeStruct((E1P, H), jnp.bfloat16)

    @functools.partial(
        pl.kernel,
        mesh=mesh,
        out_type=(shp, shp),
        scratch_types=[
            pltpu.VMEM((GNB, BLK), jnp.int32),
            pltpu.VMEM((2, GG * BLK, H), jnp.bfloat16),
            pltpu.SemaphoreType.DMA,
            pltpu.SemaphoreType.DMA,
            pltpu.SemaphoreType.DMA,
        ],
        compiler_params=pltpu.CompilerParams(use_tc_tiling_on_sc=False),
    )
    def gather2(tab_hbm, idxa_hbm, idxb_hbm,
                ma_hbm, mb_hbm, idx_v, rows_v, gsem0, gsem1, ssem):
        wid = lax.axis_index("s") * NSC + lax.axis_index("c")
        base = wid * GNB * BLK
        gsems = (gsem0, gsem1)

        def _one(idx_hbm, out_hbm):
            pltpu.sync_copy(idx_hbm.at[pl.ds(wid * GNB, GNB)], idx_v)

            def _issue(g, slot):
                for b in range(GG):
                    pltpu.async_copy(tab_hbm.at[idx_v.at[g * GG + b]],
                                     rows_v.at[slot, pl.ds(b * BLK, BLK)],
                                     gsems[slot])

            def _grp(g, slot):
                @pl.when(g + 1 < GNG)
                def _():
                    _issue(g + 1, 1 - slot)

                for b in range(GG):
                    pltpu.make_async_copy(
                        tab_hbm.at[idx_v.at[g * GG + b]],
                        rows_v.at[slot, pl.ds(b * BLK, BLK)],
                        gsems[slot]).wait()
                pltpu.async_copy(
                    rows_v.at[slot],
                    out_hbm.at[pl.ds(base + g * GG * BLK, GG * BLK)],
                    ssem).wait()

            _issue(0, 0)

            def body(g, carry):
                @pl.when(lax.rem(g, 2) == 0)
                def _():
                    _grp(g, 0)

                @pl.when(lax.rem(g, 2) == 1)
                def _():
                    _grp(g, 1)

                return carry

            lax.fori_loop(0, GNG, body, 0)
            for t in range(GTAIL):
                j = GNG * GG + t
                pltpu.async_copy(tab_hbm.at[idx_v.at[j]],
                                 rows_v.at[0, pl.ds(t * BLK, BLK)],
                                 gsem0).wait()
                pltpu.async_copy(rows_v.at[0, pl.ds(t * BLK, BLK)],
                                 out_hbm.at[pl.ds(base + j * BLK, BLK)],
                                 ssem).wait()

        _one(idxa_hbm, ma_hbm)
        _one(idxb_hbm, mb_hbm)

    return gather2


@functools.lru_cache(maxsize=None)
def _make_gather1():
    mesh = plsc.VectorSubcoreMesh(core_axis_name="c", subcore_axis_name="s")

    @functools.partial(
        pl.kernel,
        mesh=mesh,
        out_type=jax.ShapeDtypeStruct((E1P, H), jnp.bfloat16),
        scratch_types=[
            pltpu.VMEM((GNB, BLK), jnp.int32),
            pltpu.VMEM((2, GG * BLK, H), jnp.bfloat16),
            pltpu.SemaphoreType.DMA,
            pltpu.SemaphoreType.DMA,
            pltpu.SemaphoreType.DMA,
        ],
        compiler_params=pltpu.CompilerParams(use_tc_tiling_on_sc=False),
    )
    def gather1(tab_hbm, idx_hbm, m_hbm, idx_v, rows_v, gsem0, gsem1, ssem):
        wid = lax.axis_index("s") * NSC + lax.axis_index("c")
        base = wid * GNB * BLK
        gsems = (gsem0, gsem1)

        pltpu.sync_copy(idx_hbm.at[pl.ds(wid * GNB, GNB)], idx_v)

        def _issue(g, slot):
            for b in range(GG):
                pltpu.async_copy(tab_hbm.at[idx_v.at[g * GG + b]],
                                 rows_v.at[slot, pl.ds(b * BLK, BLK)],
                                 gsems[slot])

        def _grp(g, slot):
            @pl.when(g + 1 < GNG)
            def _():
                _issue(g + 1, 1 - slot)

            for b in range(GG):
                pltpu.make_async_copy(
                    tab_hbm.at[idx_v.at[g * GG + b]],
                    rows_v.at[slot, pl.ds(b * BLK, BLK)],
                    gsems[slot]).wait()
            pltpu.async_copy(
                rows_v.at[slot],
                m_hbm.at[pl.ds(base + g * GG * BLK, GG * BLK)],
                ssem).wait()

        _issue(0, 0)

        def body(g, carry):
            @pl.when(lax.rem(g, 2) == 0)
            def _():
                _grp(g, 0)

            @pl.when(lax.rem(g, 2) == 1)
            def _():
                _grp(g, 1)

            return carry

        lax.fori_loop(0, GNG, body, 0)
        for t in range(GTAIL):
            j = GNG * GG + t
            pltpu.async_copy(tab_hbm.at[idx_v.at[j]],
                             rows_v.at[0, pl.ds(t * BLK, BLK)],
                             gsem0).wait()
            pltpu.async_copy(rows_v.at[0, pl.ds(t * BLK, BLK)],
                             m_hbm.at[pl.ds(base + j * BLK, BLK)],
                             ssem).wait()

    return gather1


NBS = E1P // (NTILE * BLK)
NTS = E1P // NTILE
SG = 14
NGS = NBS // SG
NCHUNK = H // CW // NSC


@functools.lru_cache(maxsize=None)
def _make_scatter(nmsg):
    mesh = plsc.VectorSubcoreMesh(core_axis_name="c", subcore_axis_name="s")

    @functools.partial(
        pl.kernel,
        mesh=mesh,
        out_type=jax.ShapeDtypeStruct((NOP, H), jnp.float32),
        scratch_types=[
            pltpu.VMEM_SHARED((NOP, CW), jnp.float32),
            pltpu.VMEM((2, SG * BLK, CW), jnp.float32),
            pltpu.VMEM((2, SG, BLK), jnp.int32),
            pltpu.SemaphoreType.DMA,
            pltpu.SemaphoreType.DMA,
            pltpu.SemaphoreType.DMA,
            pltpu.SemaphoreType.DMA,
            pltpu.SemaphoreType.DMA,
        ],
        compiler_params=pltpu.CompilerParams(use_tc_tiling_on_sc=False),
    )
    def _scatter(*args):
        msgs = args[:nmsg]
        idxs = args[nmsg:2 * nmsg]
        (init_hbm, agg_hbm, acc, mbuf, ibuf,
         isem0, isem1, rsem0, rsem1, asem) = args[2 * nmsg:]
        sc = lax.axis_index("c")
        tid = lax.axis_index("s")
        isems = (isem0, isem1)
        rsems = (rsem0, rsem1)

        def _phase(msg_hbm, idx_hbm, src_col0):
            def _refs(g, slot):
                isrc = idx_hbm.at[pl.ds(tid * NBS + g * SG, SG)]
                msrc = msg_hbm.at[pl.ds(tid * NTS + g * SG * BLK, SG * BLK),
                                  pl.ds(src_col0, CW)]
                return ((isrc, ibuf.at[slot], isems[slot]),
                        (msrc, mbuf.at[slot], rsems[slot]))

            def _issue(g, slot):
                for src, dst, sem in _refs(g, slot):
                    pltpu.async_copy(src, dst, sem)

            def _grp(g, slot):
                @pl.when(g + 1 < NGS)
                def _():
                    _issue(g + 1, 1 - slot)

                for src, dst, sem in _refs(g, slot):
                    pltpu.make_async_copy(src, dst, sem).wait()
                ads = [pltpu.async_copy(
                           mbuf.at[slot, pl.ds(b * BLK, BLK)],
                           acc.at[ibuf.at[slot, b]], asem, add=True)
                       for b in range(SG)]
                for d in ads:
                    d.wait()

            _issue(0, 0)

            def body(g, carry):
                @pl.when(lax.rem(g, 2) == 0)
                def _():
                    _grp(g, 0)

                @pl.when(lax.rem(g, 2) == 1)
                def _():
                    _grp(g, 1)

                return carry

            lax.fori_loop(0, NGS, body, 0)

        for c_local in range(NCHUNK):
            col0 = (sc * NCHUNK + c_local) * CW
            pltpu.sync_copy(init_hbm.at[pl.ds(tid * STRIPE, STRIPE),
                                        pl.ds(col0, CW)],
                            acc.at[pl.ds(tid * STRIPE, STRIPE)])
            plsc.subcore_barrier()
            for m, i in zip(msgs, idxs):
                _phase(m, i, col0)
            plsc.subcore_barrier()
            pltpu.sync_copy(acc.at[pl.ds(tid * STRIPE, STRIPE)],
                            agg_hbm.at[pl.ds(tid * STRIPE, STRIPE),
                                       pl.ds(col0, CW)])

    return _scatter


def kernel(x_obj, x_p1, x_p2,
           eo_W1, eo_b1, eo_W2, eo_b2,
           e1_W1, e1_b1, e1_W2, e1_b2,
           e2_W1, e2_b1, e2_W2, e2_b2,
           a1_W1, a1_b1, a1_W2, a1_b2,
           a2_W1, a2_b1, a2_W2, a2_b2,
           u_W1, u_b1, u_W2, u_b2,
           edge_index_p1, edge_index_p2, batch_obj):
    f32 = jnp.float32

    x_obj_p = jnp.pad(x_obj, ((0, NOP - N_OBJ), (0, 0)))
    x_p1_p = jnp.pad(x_p1, ((0, P1P - N_P1), (0, 0)))
    x_p2_p = jnp.pad(x_p2, ((0, P2P - N_P2), (0, 0)))

    trash1 = N_OBJ + (jnp.arange(E1P - N_P1, dtype=jnp.int32) % (NOP - N_OBJ))
    trash2 = N_OBJ + (jnp.arange(E2P - 2 * N_P2, dtype=jnp.int32)
                      % (NOP - N_OBJ))
    src1 = jnp.concatenate([edge_index_p1[0], trash1]).reshape(E1P // BLK, BLK)
    s2 = jnp.concatenate([edge_index_p2[0], trash2]).reshape(P2P, 2)
    src2e = s2[:, 0].reshape(P2P // BLK, BLK)
    src2o = s2[:, 1].reshape(P2P // BLK, BLK)

    bid3d = jnp.pad(batch_obj, (0, NOP - N_OBJ),
                    constant_values=NUM_GRAPHS).reshape(NOP // 1024, 1, 1024)
    zeros_agg = jnp.zeros((NOP, H), f32)

    h_obj, h_obj_bf = _mlp_obj(x_obj_p, eo_W1, eo_b1, eo_W2, eo_b2, bn=1024)
    h_p1 = _mlp(x_p1_p, e1_W1, e1_b1, e1_W2, e1_b2, bn=1024)
    h_p2a, h_p2b = _mlp_split(x_p2_p, e2_W1, e2_b1, e2_W2, e2_b2, bn=1024)

    u_W1a = u_W1[:H]
    u_W1b = u_W1[H:]

    gather1 = _make_gather1()
    gather2 = _make_gather()
    scatter1 = _make_scatter(1)
    scatter2 = _make_scatter(2)

    for _ in range(NUM_LAYER):
        m1 = gather1(h_obj_bf, src1)
        m2a, m2b = gather2(h_obj_bf, src2e, src2o)
        h_p1 = _mlp_add(h_p1, m1, a1_W1, a1_b1, a1_W2, a1_b2, bn=1024)
        h_p2a, h_p2b = _mlp_add2(h_p2a, h_p2b, m2a, m2b,
                                 a2_W1, a2_b1, a2_W2, a2_b2, bn=1024)
        agg1 = scatter1(h_p1, src1, zeros_agg)
        agg = scatter2(h_p2a, h_p2b, src2e, src2o, agg1)
        h_obj, h_obj_bf = _mlp_cat(h_obj, agg, u_W1a, u_W1b, u_b1,
                                   u_W2, u_b2, bn=1024)

    return _pool(h_obj, bid3d, bn=1024)


# speedup vs baseline: 1.3989x; 1.3989x over previous
import functools

import jax
import jax.numpy as jnp
from jax import lax
from jax.experimental import pallas as pl
from jax.experimental.pallas import tpu as pltpu
from jax.experimental.pallas import tpu_sc as plsc

H = 128
NUM_LAYER = 2
NUM_GRAPHS = 64

N_OBJ = 50000
N_P1 = 200000
N_P2 = 200000

NOP = 50176
P1P = 200704
P2P = 200704
E1P = P1P
E2P = 2 * P2P

NW = 32
NSC = 2
NTILE = 16
BLK = 128
CW = 16
STRIPE = NOP // NTILE


def _mlp_body(x_ref, w1_ref, b1_ref, w2_ref, b2_ref, o_ref):
    x = x_ref[...]
    h = jnp.maximum(
        jnp.dot(x, w1_ref[...], preferred_element_type=jnp.float32)
        + b1_ref[...], 0.0)
    o_ref[...] = (
        jnp.dot(h, w2_ref[...], preferred_element_type=jnp.float32)
        + b2_ref[...])


def _mlp_add_body(x1_ref, x2_ref, w1_ref, b1_ref, w2_ref, b2_ref, o_ref):
    x = x1_ref[...] + x2_ref[...]
    h = jnp.maximum(
        jnp.dot(x, w1_ref[...], preferred_element_type=jnp.float32)
        + b1_ref[...], 0.0)
    o_ref[...] = (
        jnp.dot(h, w2_ref[...], preferred_element_type=jnp.float32)
        + b2_ref[...])


def _mlp_split_body(x_ref, w1_ref, b1_ref, w2_ref, b2_ref, oa_ref, ob_ref):
    x = x_ref[...]
    h = jnp.maximum(
        jnp.dot(x, w1_ref[...], preferred_element_type=jnp.float32)
        + b1_ref[...], 0.0)
    o = (jnp.dot(h, w2_ref[...], preferred_element_type=jnp.float32)
         + b2_ref[...])
    oa_ref[...] = o[:, :H]
    ob_ref[...] = o[:, H:]


def _mlp_add2_body(x1a_ref, x1b_ref, x2a_ref, x2b_ref, w1_ref, b1_ref,
                   w2_ref, b2_ref, oa_ref, ob_ref):
    x = jnp.concatenate(
        [x1a_ref[...] + x2a_ref[...], x1b_ref[...] + x2b_ref[...]], axis=1)
    h = jnp.maximum(
        jnp.dot(x, w1_ref[...], preferred_element_type=jnp.float32)
        + b1_ref[...], 0.0)
    o = (jnp.dot(h, w2_ref[...], preferred_element_type=jnp.float32)
         + b2_ref[...])
    oa_ref[...] = o[:, :H]
    ob_ref[...] = o[:, H:]


def _mlp_cat_body(x1_ref, x2_ref, w1a_ref, w1b_ref, b1_ref, w2_ref, b2_ref,
                  o_ref):
    z = (jnp.dot(x1_ref[...], w1a_ref[...], preferred_element_type=jnp.float32)
         + jnp.dot(x2_ref[...], w1b_ref[...],
                   preferred_element_type=jnp.float32)
         + b1_ref[...])
    h = jnp.maximum(z, 0.0)
    o_ref[...] = (
        jnp.dot(h, w2_ref[...], preferred_element_type=jnp.float32)
        + b2_ref[...])


def _const_spec(shape):
    return pl.BlockSpec(shape, lambda i: (0,) * len(shape))


def _mlp(x, w1, b1, w2, b2, bn):
    n, k = x.shape
    h1 = w1.shape[1]
    h2 = w2.shape[1]
    return pl.pallas_call(
        _mlp_body,
        grid=(n // bn,),
        in_specs=[
            pl.BlockSpec((bn, k), lambda i: (i, 0)),
            _const_spec((k, h1)),
            _const_spec((1, h1)),
            _const_spec((h1, h2)),
            _const_spec((1, h2)),
        ],
        out_specs=pl.BlockSpec((bn, h2), lambda i: (i, 0)),
        out_shape=jax.ShapeDtypeStruct((n, h2), jnp.float32),
    )(x, w1, b1.reshape(1, -1), w2, b2.reshape(1, -1))


def _mlp_add(x1, x2, w1, b1, w2, b2, bn):
    n, k = x1.shape
    h1 = w1.shape[1]
    h2 = w2.shape[1]
    return pl.pallas_call(
        _mlp_add_body,
        grid=(n // bn,),
        in_specs=[
            pl.BlockSpec((bn, k), lambda i: (i, 0)),
            pl.BlockSpec((bn, k), lambda i: (i, 0)),
            _const_spec((k, h1)),
            _const_spec((1, h1)),
            _const_spec((h1, h2)),
            _const_spec((1, h2)),
        ],
        out_specs=pl.BlockSpec((bn, h2), lambda i: (i, 0)),
        out_shape=jax.ShapeDtypeStruct((n, h2), jnp.float32),
    )(x1, x2, w1, b1.reshape(1, -1), w2, b2.reshape(1, -1))


def _mlp_split(x, w1, b1, w2, b2, bn):
    n, k = x.shape
    h1 = w1.shape[1]
    return pl.pallas_call(
        _mlp_split_body,
        grid=(n // bn,),
        in_specs=[
            pl.BlockSpec((bn, k), lambda i: (i, 0)),
            _const_spec((k, h1)),
            _const_spec((1, h1)),
            _const_spec((h1, 2 * H)),
            _const_spec((1, 2 * H)),
        ],
        out_specs=[pl.BlockSpec((bn, H), lambda i: (i, 0)),
                   pl.BlockSpec((bn, H), lambda i: (i, 0))],
        out_shape=[jax.ShapeDtypeStruct((n, H), jnp.float32),
                   jax.ShapeDtypeStruct((n, H), jnp.float32)],
    )(x, w1, b1.reshape(1, -1), w2, b2.reshape(1, -1))


def _mlp_add2(x1a, x1b, x2a, x2b, w1, b1, w2, b2, bn):
    n = x1a.shape[0]
    h1 = w1.shape[1]
    return pl.pallas_call(
        _mlp_add2_body,
        grid=(n // bn,),
        in_specs=[
            pl.BlockSpec((bn, H), lambda i: (i, 0)),
            pl.BlockSpec((bn, H), lambda i: (i, 0)),
            pl.BlockSpec((bn, H), lambda i: (i, 0)),
            pl.BlockSpec((bn, H), lambda i: (i, 0)),
            _const_spec((2 * H, h1)),
            _const_spec((1, h1)),
            _const_spec((h1, 2 * H)),
            _const_spec((1, 2 * H)),
        ],
        out_specs=[pl.BlockSpec((bn, H), lambda i: (i, 0)),
                   pl.BlockSpec((bn, H), lambda i: (i, 0))],
        out_shape=[jax.ShapeDtypeStruct((n, H), jnp.float32),
                   jax.ShapeDtypeStruct((n, H), jnp.float32)],
    )(x1a, x1b, x2a, x2b, w1, b1.reshape(1, -1), w2, b2.reshape(1, -1))


def _mlp_cat(x1, x2, w1a, w1b, b1, w2, b2, bn):
    n, k = x1.shape
    h1 = w1a.shape[1]
    h2 = w2.shape[1]
    return pl.pallas_call(
        _mlp_cat_body,
        grid=(n // bn,),
        in_specs=[
            pl.BlockSpec((bn, k), lambda i: (i, 0)),
            pl.BlockSpec((bn, k), lambda i: (i, 0)),
            _const_spec((k, h1)),
            _const_spec((k, h1)),
            _const_spec((1, h1)),
            _const_spec((h1, h2)),
            _const_spec((1, h2)),
        ],
        out_specs=pl.BlockSpec((bn, h2), lambda i: (i, 0)),
        out_shape=jax.ShapeDtypeStruct((n, h2), jnp.float32),
    )(x1, x2, w1a, w1b, b1.reshape(1, -1), w2, b2.reshape(1, -1))


def _pool_body(x_ref, bid_ref, o_ref):
    ids = bid_ref[0, 0, :]
    bn = ids.shape[0]
    onehot = (ids.reshape(bn, 1)
              == lax.broadcasted_iota(jnp.int32, (1, NUM_GRAPHS), 1)
              ).astype(jnp.float32)
    contrib = lax.dot_general(
        onehot, x_ref[...], (((0,), (0,)), ((), ())),
        preferred_element_type=jnp.float32)

    @pl.when(pl.program_id(0) == 0)
    def _():
        o_ref[...] = jnp.zeros_like(o_ref)

    o_ref[...] += contrib


def _pool(x, bid3d, bn):
    n = x.shape[0]
    return pl.pallas_call(
        _pool_body,
        grid=(n // bn,),
        in_specs=[
            pl.BlockSpec((bn, H), lambda i: (i, 0)),
            pl.BlockSpec((1, 1, bn), lambda i: (i, 0, 0)),
        ],
        out_specs=pl.BlockSpec((NUM_GRAPHS, H), lambda i: (0, 0)),
        out_shape=jax.ShapeDtypeStruct((NUM_GRAPHS, H), jnp.float32),
    )(x, bid3d)


GG = 7


@functools.lru_cache(maxsize=None)
def _make_gather(n_edges):
    nb = n_edges // (NW * BLK)
    ng = nb // GG
    mesh = plsc.VectorSubcoreMesh(core_axis_name="c", subcore_axis_name="s")

    @functools.partial(
        pl.kernel,
        mesh=mesh,
        out_type=jax.ShapeDtypeStruct((n_edges, H), jnp.float32),
        scratch_types=[
            pltpu.VMEM((nb, BLK), jnp.int32),
            pltpu.VMEM((GG, BLK, H), jnp.float32),
            pltpu.SemaphoreType.DMA,
            pltpu.SemaphoreType.DMA,
        ],
        compiler_params=pltpu.CompilerParams(use_tc_tiling_on_sc=False),
    )
    def gather(tab_hbm, idx_hbm, out_hbm, idx_v, rows_v, gsem, ssem):
        wid = lax.axis_index("s") * NSC + lax.axis_index("c")
        pltpu.sync_copy(idx_hbm.at[pl.ds(wid * nb, nb)], idx_v)
        base = wid * nb * BLK

        def grp(g, carry):
            gds = [pltpu.async_copy(tab_hbm.at[idx_v.at[g * GG + b]],
                                    rows_v.at[b], gsem)
                   for b in range(GG)]
            for d in gds:
                d.wait()
            sds = [pltpu.async_copy(
                       rows_v.at[b],
                       out_hbm.at[pl.ds(base + (g * GG + b) * BLK, BLK)],
                       ssem)
                   for b in range(GG)]
            for d in sds:
                d.wait()
            return carry

        lax.fori_loop(0, ng, grp, 0)

    return gather


NBS = E1P // (NTILE * BLK)
NTS = E1P // NTILE
SG = 14
NGS = NBS // SG
NCHUNK = H // CW // NSC


@functools.lru_cache(maxsize=None)
def _make_scatter():
    mesh = plsc.VectorSubcoreMesh(core_axis_name="c", subcore_axis_name="s")

    @functools.partial(
        pl.kernel,
        mesh=mesh,
        out_type=jax.ShapeDtypeStruct((NOP, H), jnp.float32),
        scratch_types=[
            pltpu.VMEM_SHARED((NOP, CW), jnp.float32),
            pltpu.VMEM((2, SG * BLK, CW), jnp.float32),
            pltpu.VMEM((2, SG, BLK), jnp.int32),
            pltpu.SemaphoreType.DMA,
            pltpu.SemaphoreType.DMA,
            pltpu.SemaphoreType.DMA,
            pltpu.SemaphoreType.DMA,
            pltpu.SemaphoreType.DMA,
        ],
        compiler_params=pltpu.CompilerParams(use_tc_tiling_on_sc=False),
    )
    def _scatter(msg1_hbm, msg2a_hbm, msg2b_hbm, idx1_hbm, idx2e_hbm,
                 idx2o_hbm, zero_hbm, agg_hbm, acc, mbuf, ibuf,
                 isem0, isem1, rsem0, rsem1, asem):
        sc = lax.axis_index("c")
        tid = lax.axis_index("s")
        isems = (isem0, isem1)
        rsems = (rsem0, rsem1)

        def _phase(msg_hbm, idx_hbm, src_col0):
            def _refs(g, slot):
                isrc = idx_hbm.at[pl.ds(tid * NBS + g * SG, SG)]
                msrc = msg_hbm.at[pl.ds(tid * NTS + g * SG * BLK, SG * BLK),
                                  pl.ds(src_col0, CW)]
                return ((isrc, ibuf.at[slot], isems[slot]),
                        (msrc, mbuf.at[slot], rsems[slot]))

            def _issue(g, slot):
                for src, dst, sem in _refs(g, slot):
                    pltpu.async_copy(src, dst, sem)

            def _grp(g, slot):
                @pl.when(g + 1 < NGS)
                def _():
                    _issue(g + 1, 1 - slot)

                for src, dst, sem in _refs(g, slot):
                    pltpu.make_async_copy(src, dst, sem).wait()
                ads = [pltpu.async_copy(
                           mbuf.at[slot, pl.ds(b * BLK, BLK)],
                           acc.at[ibuf.at[slot, b]], asem, add=True)
                       for b in range(SG)]
                for d in ads:
                    d.wait()

            _issue(0, 0)

            def body(g, carry):
                @pl.when(lax.rem(g, 2) == 0)
                def _():
                    _grp(g, 0)

                @pl.when(lax.rem(g, 2) == 1)
                def _():
                    _grp(g, 1)

                return carry

            lax.fori_loop(0, NGS, body, 0)

        for c_local in range(NCHUNK):
            col0 = (sc * NCHUNK + c_local) * CW
            pltpu.sync_copy(zero_hbm.at[pl.ds(tid * STRIPE, STRIPE)],
                            acc.at[pl.ds(tid * STRIPE, STRIPE)])
            plsc.subcore_barrier()
            _phase(msg1_hbm, idx1_hbm, col0)
            _phase(msg2a_hbm, idx2e_hbm, col0)
            _phase(msg2b_hbm, idx2o_hbm, col0)
            plsc.subcore_barrier()
            pltpu.sync_copy(acc.at[pl.ds(tid * STRIPE, STRIPE)],
                            agg_hbm.at[pl.ds(tid * STRIPE, STRIPE),
                                       pl.ds(col0, CW)])

    return _scatter


def kernel(x_obj, x_p1, x_p2,
           eo_W1, eo_b1, eo_W2, eo_b2,
           e1_W1, e1_b1, e1_W2, e1_b2,
           e2_W1, e2_b1, e2_W2, e2_b2,
           a1_W1, a1_b1, a1_W2, a1_b2,
           a2_W1, a2_b1, a2_W2, a2_b2,
           u_W1, u_b1, u_W2, u_b2,
           edge_index_p1, edge_index_p2, batch_obj):
    f32 = jnp.float32

    x_obj_p = jnp.pad(x_obj, ((0, NOP - N_OBJ), (0, 0)))
    x_p1_p = jnp.pad(x_p1, ((0, P1P - N_P1), (0, 0)))
    x_p2_p = jnp.pad(x_p2, ((0, P2P - N_P2), (0, 0)))

    trash1 = N_OBJ + (jnp.arange(E1P - N_P1, dtype=jnp.int32) % (NOP - N_OBJ))
    trash2 = N_OBJ + (jnp.arange(E2P - 2 * N_P2, dtype=jnp.int32)
                      % (NOP - N_OBJ))
    src1 = jnp.concatenate([edge_index_p1[0], trash1]).reshape(E1P // BLK, BLK)
    s2 = jnp.concatenate([edge_index_p2[0], trash2]).reshape(P2P, 2)
    src2e = s2[:, 0].reshape(P2P // BLK, BLK)
    src2o = s2[:, 1].reshape(P2P // BLK, BLK)

    bid3d = jnp.pad(batch_obj, (0, NOP - N_OBJ),
                    constant_values=NUM_GRAPHS).reshape(NOP // 1024, 1, 1024)
    zeros_chunk = jnp.zeros((NOP, CW), f32)

    h_obj = _mlp(x_obj_p, eo_W1, eo_b1, eo_W2, eo_b2, bn=1024)
    h_p1 = _mlp(x_p1_p, e1_W1, e1_b1, e1_W2, e1_b2, bn=1024)
    h_p2a, h_p2b = _mlp_split(x_p2_p, e2_W1, e2_b1, e2_W2, e2_b2, bn=1024)

    u_W1a = u_W1[:H]
    u_W1b = u_W1[H:]

    gather = _make_gather(E1P)
    scatter = _make_scatter()

    for _ in range(NUM_LAYER):
        m1 = gather(h_obj, src1)
        m2a = gather(h_obj, src2e)
        m2b = gather(h_obj, src2o)
        h_p1 = _mlp_add(h_p1, m1, a1_W1, a1_b1, a1_W2, a1_b2, bn=1024)
        h_p2a, h_p2b = _mlp_add2(h_p2a, h_p2b, m2a, m2b,
                                 a2_W1, a2_b1, a2_W2, a2_b2, bn=1024)
        agg = scatter(h_p1, h_p2a, h_p2b, src1, src2e, src2o, zeros_chunk)
        h_obj = _mlp_cat(h_obj, agg, u_W1a, u_W1b, u_b1, u_W2, u_b2, bn=1024)

    return _pool(h_obj, bid3d, bn=1024)
